# baseline (speedup 1.0000x reference)
import functools
import numpy as np
import jax
import jax.numpy as jnp
from jax.experimental import pallas as pl

_CLASSES = 80
_NMS_T = 0.6
_PRE = 256
_MAXB = 100
_ANCH = np.array([[12, 16], [19, 36], [40, 28], [36, 75], [76, 55],
                  [72, 146], [142, 110], [192, 243], [459, 401]],
                 dtype=np.float32)
_LEVELS = (
    (64, 64, (0, 1, 2), 8.0, 1.2),
    (32, 32, (3, 4, 5), 16.0, 1.1),
    (16, 16, (6, 7, 8), 32.0, 1.05),
)
_NTOT = 16128
_NCH = 126
_FCH = 160


def _sig(x):
    return 1.0 / (1.0 + jnp.exp(-x))


def _decode_body(x_ref, s_ref, b_ref, *, H, W, anchors, stride, scale):
    HW = H * W
    ipos = jax.lax.broadcasted_iota(jnp.int32, (1, HW), 1)
    gx = (ipos % W).astype(jnp.float32)
    gy = (ipos // W).astype(jnp.float32)
    bias = 0.5 * (scale - 1.0)
    for a in range(3):
        base = a * 85
        head = x_ref[0, base:base + 5, :]
        cls = x_ref[0, base + 5:base + 85, :]
        tx, ty = head[0:1], head[1:2]
        tw, th = head[2:3], head[3:4]
        obj = head[4:5]
        xc = (_sig(tx) * scale - bias + gx) / W
        yc = (_sig(ty) * scale - bias + gy) / H
        w = jnp.exp(tw) * (anchors[a][0] / (stride * W))
        h = jnp.exp(th) * (anchors[a][1] / (stride * H))
        b_ref[0, :, a, :] = jnp.concatenate(
            [yc - h * 0.5, xc - w * 0.5, yc + h * 0.5, xc + w * 0.5], axis=0)
        s_ref[0, :, a, :] = _sig(cls) * _sig(obj)


def _decode_level(x, H, W, mask, stride, scale):
    B = x.shape[0]
    HW = H * W
    xr = jnp.transpose(x.reshape(B, HW, 255), (0, 2, 1))
    anchors = tuple((float(_ANCH[m, 0]), float(_ANCH[m, 1])) for m in mask)
    body = functools.partial(_decode_body, H=H, W=W, anchors=anchors,
                             stride=stride, scale=scale)
    s, b = pl.pallas_call(
        body,
        grid=(B,),
        in_specs=[pl.BlockSpec((1, 255, HW), lambda i: (i, 0, 0))],
        out_specs=[pl.BlockSpec((1, _CLASSES, 3, HW), lambda i: (i, 0, 0, 0)),
                   pl.BlockSpec((1, 4, 3, HW), lambda i: (i, 0, 0, 0))],
        out_shape=[jax.ShapeDtypeStruct((B, _CLASSES, 3, HW), jnp.float32),
                   jax.ShapeDtypeStruct((B, 4, 3, HW), jnp.float32)],
    )(xr)
    s = jnp.transpose(s, (0, 1, 3, 2)).reshape(B, _CLASSES, HW * 3)
    b = jnp.transpose(b, (0, 1, 3, 2)).reshape(B, 4, HW * 3)
    return s, b


_G = 4


def _nms_body(s_ref, bx_ref, os_ref, ob_ref):
    i126 = jax.lax.broadcasted_iota(jnp.int32, (1, _NCH), 1)
    i128 = jax.lax.broadcasted_iota(jnp.int32, (1, 128), 1)
    i256 = jax.lax.broadcasted_iota(jnp.int32, (1, _PRE), 1)
    z256 = jnp.zeros((1, _PRE), jnp.float32)

    init = tuple(
        (jnp.max(s_ref[0, g], axis=1).reshape(1, _NCH),
         z256, z256, z256, z256, z256)
        for g in range(_G))

    def pick(k, carry):
        out = []
        for g in range(_G):
            cmax, ts, b0, b1, b2, b3 = carry[g]
            m = jnp.max(cmax)
            cid = jnp.min(jnp.where(cmax == m, i126, _NCH))
            row = s_ref[0, g, pl.ds(cid, 1), :]
            lane = jnp.min(jnp.where(row == m, i128, 128))
            sel = i128 == lane

            def getb(j, cid=cid, sel=sel):
                br = bx_ref[0, j, pl.ds(cid, 1), :]
                return jnp.sum(jnp.where(sel, br, 0.0))

            upd = i256 == k
            ts = jnp.where(upd, m, ts)
            b0 = jnp.where(upd, getb(0), b0)
            b1 = jnp.where(upd, getb(1), b1)
            b2 = jnp.where(upd, getb(2), b2)
            b3 = jnp.where(upd, getb(3), b3)
            nrow = jnp.where(sel, -1.0, row)
            s_ref[0, g, pl.ds(cid, 1), :] = nrow
            cmax = jnp.where(i126 == cid, jnp.max(nrow), cmax)
            out.append((cmax, ts, b0, b1, b2, b3))
        return tuple(out)

    picked = jax.lax.fori_loop(0, _PRE, pick, init)
    areas = [jnp.maximum(p[4] - p[2], 0.0) * jnp.maximum(p[5] - p[3], 0.0)
             for p in picked]

    def nms(i, keeps):
        sel = i256 == i
        out = []
        for g in range(_G):
            _, ts, b0, b1, b2, b3 = picked[g]
            keep = keeps[g]
            y0 = jnp.sum(jnp.where(sel, b0, 0.0))
            x0 = jnp.sum(jnp.where(sel, b1, 0.0))
            y1 = jnp.sum(jnp.where(sel, b2, 0.0))
            x1 = jnp.sum(jnp.where(sel, b3, 0.0))
            ai = jnp.maximum(y1 - y0, 0.0) * jnp.maximum(x1 - x0, 0.0)
            ih = jnp.maximum(jnp.minimum(b2, y1) - jnp.maximum(b0, y0), 0.0)
            iw = jnp.maximum(jnp.minimum(b3, x1) - jnp.maximum(b1, x0), 0.0)
            inter = ih * iw
            iou = inter / jnp.maximum(areas[g] + ai - inter, 1e-9)
            ki = jnp.sum(jnp.where(sel, keep, 0.0))
            out.append(jnp.where((i256 > i) & (iou > _NMS_T) & (ki > 0.5),
                                 0.0, keep))
        return tuple(out)

    keeps = jax.lax.fori_loop(0, _PRE, nms,
                              tuple(jnp.ones((1, _PRE), jnp.float32)
                                    for _ in range(_G)))
    for g in range(_G):
        _, ts, b0, b1, b2, b3 = picked[g]
        os_ref[0, g] = ts * keeps[g]
        ob_ref[0, g] = jnp.concatenate([b0, b1, b2, b3], axis=0)


def _final_body(s_ref, b_ref, fb_ref, fs_ref, fc_ref, v_ref):
    i160 = jax.lax.broadcasted_iota(jnp.int32, (_FCH, 1), 0)
    i128 = jax.lax.broadcasted_iota(jnp.int32, (1, 128), 1)
    i100 = jax.lax.broadcasted_iota(jnp.int32, (1, _MAXB), 1)
    z100 = jnp.zeros((1, _MAXB), jnp.float32)
    cmax0 = jnp.max(s_ref[0], axis=1, keepdims=True)

    def pick(k, carry):
        cmax, fs, f0, f1, f2, f3, fc = carry
        m = jnp.max(cmax)
        cid = jnp.min(jnp.where(cmax == m, i160, _FCH))
        row = s_ref[0, pl.ds(cid, 1), :]
        lane = jnp.min(jnp.where(row == m, i128, 128))
        sel = i128 == lane
        flat = cid * 128 + lane
        cls = (flat // _PRE).astype(jnp.float32)

        def getb(j):
            br = b_ref[0, j, pl.ds(cid, 1), :]
            return jnp.sum(jnp.where(sel, br, 0.0))

        upd = i100 == k
        fs = jnp.where(upd, m, fs)
        f0 = jnp.where(upd, getb(0), f0)
        f1 = jnp.where(upd, getb(1), f1)
        f2 = jnp.where(upd, getb(2), f2)
        f3 = jnp.where(upd, getb(3), f3)
        fc = jnp.where(upd, cls, fc)
        nrow = jnp.where(sel, -1.0, row)
        s_ref[0, pl.ds(cid, 1), :] = nrow
        cmax = jnp.where(i160 == cid, jnp.max(nrow), cmax)
        return cmax, fs, f0, f1, f2, f3, fc

    init = (cmax0, z100, z100, z100, z100, z100, z100)
    _, fs, f0, f1, f2, f3, fc = jax.lax.fori_loop(0, _MAXB, pick, init)
    fs_ref[0] = fs
    fc_ref[0] = fc
    fb_ref[0] = jnp.concatenate([f0, f1, f2, f3], axis=0)
    v_ref[0] = jnp.sum((fs > 0.0).astype(jnp.int32), axis=1, keepdims=True)


def kernel(inputs_3, inputs_4, inputs_5):
    B = inputs_3.shape[0]
    xs = (inputs_3, inputs_4, inputs_5)
    s_all, b_all = [], []
    for x, (H, W, mask, stride, scale) in zip(xs, _LEVELS):
        s, b = _decode_level(x, H, W, mask, stride, scale)
        s_all.append(s)
        b_all.append(b)
    scores = jnp.concatenate(s_all, axis=2)
    boxes = jnp.concatenate(b_all, axis=2)
    scores = scores.reshape(B, _CLASSES, _NCH, 128)
    boxes = boxes.reshape(B, 4, _NCH, 128)

    out_s, out_b = pl.pallas_call(
        _nms_body,
        grid=(B, _CLASSES // _G),
        in_specs=[
            pl.BlockSpec((1, _G, _NCH, 128), lambda b, c: (b, c, 0, 0)),
            pl.BlockSpec((1, 4, _NCH, 128), lambda b, c: (b, 0, 0, 0)),
        ],
        out_specs=[
            pl.BlockSpec((1, _G, 1, _PRE), lambda b, c: (b, c, 0, 0)),
            pl.BlockSpec((1, _G, 4, _PRE), lambda b, c: (b, c, 0, 0)),
        ],
        out_shape=[
            jax.ShapeDtypeStruct((B, _CLASSES, 1, _PRE), jnp.float32),
            jax.ShapeDtypeStruct((B, _CLASSES, 4, _PRE), jnp.float32),
        ],
    )(scores, boxes)

    flat_s = out_s.reshape(B, _FCH, 128)
    flat_b = jnp.transpose(out_b, (0, 2, 1, 3)).reshape(B, 4, _FCH, 128)

    fb, fs, fc, valid = pl.pallas_call(
        _final_body,
        grid=(B,),
        in_specs=[
            pl.BlockSpec((1, _FCH, 128), lambda i: (i, 0, 0)),
            pl.BlockSpec((1, 4, _FCH, 128), lambda i: (i, 0, 0, 0)),
        ],
        out_specs=[
            pl.BlockSpec((1, 4, _MAXB), lambda i: (i, 0, 0)),
            pl.BlockSpec((1, 1, _MAXB), lambda i: (i, 0, 0)),
            pl.BlockSpec((1, 1, _MAXB), lambda i: (i, 0, 0)),
            pl.BlockSpec((1, 1, 1), lambda i: (i, 0, 0)),
        ],
        out_shape=[
            jax.ShapeDtypeStruct((B, 4, _MAXB), jnp.float32),
            jax.ShapeDtypeStruct((B, 1, _MAXB), jnp.float32),
            jax.ShapeDtypeStruct((B, 1, _MAXB), jnp.float32),
            jax.ShapeDtypeStruct((B, 1, 1), jnp.int32),
        ],
    )(flat_s, flat_b)

    fin_b = jnp.transpose(fb, (0, 2, 1))
    fin_s = fs.reshape(B, _MAXB)
    fin_c = fc.reshape(B, _MAXB)
    return fin_b, fin_s, fin_c, valid.reshape(B)


# speedup vs baseline: 2.0854x; 2.0854x over previous
import functools
import numpy as np
import jax
import jax.numpy as jnp
from jax.experimental import pallas as pl
from jax.experimental.pallas import tpu as pltpu

_CLASSES = 80
_NMS_T = 0.6
_PRE = 256
_MAXB = 100
_ANCH = np.array([[12, 16], [19, 36], [40, 28], [36, 75], [76, 55],
                  [72, 146], [142, 110], [192, 243], [459, 401]],
                 dtype=np.float32)
_LEVELS = (
    (64, 64, (0, 1, 2), 8.0, 1.2),
    (32, 32, (3, 4, 5), 16.0, 1.1),
    (16, 16, (6, 7, 8), 32.0, 1.05),
)
_NTOT = 16128
_NCH = 126
_FCH = 160


def _sig(x):
    return 1.0 / (1.0 + jnp.exp(-x))


def _decode_body(x_ref, s_ref, b_ref, *, H, W, anchors, stride, scale):
    HW = H * W
    ipos = jax.lax.broadcasted_iota(jnp.int32, (1, HW), 1)
    gx = (ipos % W).astype(jnp.float32)
    gy = (ipos // W).astype(jnp.float32)
    bias = 0.5 * (scale - 1.0)
    for a in range(3):
        base = a * 85
        head = x_ref[0, base:base + 5, :]
        cls = x_ref[0, base + 5:base + 85, :]
        tx, ty = head[0:1], head[1:2]
        tw, th = head[2:3], head[3:4]
        obj = head[4:5]
        xc = (_sig(tx) * scale - bias + gx) / W
        yc = (_sig(ty) * scale - bias + gy) / H
        w = jnp.exp(tw) * (anchors[a][0] / (stride * W))
        h = jnp.exp(th) * (anchors[a][1] / (stride * H))
        b_ref[0, :, a, :] = jnp.concatenate(
            [yc - h * 0.5, xc - w * 0.5, yc + h * 0.5, xc + w * 0.5], axis=0)
        s_ref[0, :, a, :] = _sig(cls) * _sig(obj)


def _decode_level(x, H, W, mask, stride, scale):
    B = x.shape[0]
    HW = H * W
    xr = jnp.transpose(x.reshape(B, HW, 255), (0, 2, 1))
    anchors = tuple((float(_ANCH[m, 0]), float(_ANCH[m, 1])) for m in mask)
    body = functools.partial(_decode_body, H=H, W=W, anchors=anchors,
                             stride=stride, scale=scale)
    s, b = pl.pallas_call(
        body,
        grid=(B,),
        in_specs=[pl.BlockSpec((1, 255, HW), lambda i: (i, 0, 0))],
        out_specs=[pl.BlockSpec((1, _CLASSES, 3, HW), lambda i: (i, 0, 0, 0)),
                   pl.BlockSpec((1, 4, 3, HW), lambda i: (i, 0, 0, 0))],
        out_shape=[jax.ShapeDtypeStruct((B, _CLASSES, 3, HW), jnp.float32),
                   jax.ShapeDtypeStruct((B, 4, 3, HW), jnp.float32)],
    )(xr)
    s = jnp.transpose(s, (0, 1, 3, 2)).reshape(B, _CLASSES, HW * 3)
    b = jnp.transpose(b, (0, 1, 3, 2)).reshape(B, 4, HW * 3)
    return s, b


_G = 2


def _nms_body(s_ref, bx_ref, os_ref, ob_ref, iou_ref):
    i126 = jax.lax.broadcasted_iota(jnp.int32, (1, _NCH), 1)
    i128 = jax.lax.broadcasted_iota(jnp.int32, (1, 128), 1)
    i256 = jax.lax.broadcasted_iota(jnp.int32, (1, _PRE), 1)
    iflat = (jax.lax.broadcasted_iota(jnp.int32, (_NCH, 128), 0) * 128
             + jax.lax.broadcasted_iota(jnp.int32, (_NCH, 128), 1))
    z256f = jnp.zeros((1, _PRE), jnp.float32)
    z256i = jnp.zeros((1, _PRE), jnp.int32)

    init = tuple((s_ref[0, g], z256f, z256i) for g in range(_G))

    def pick(k, carry):
        out = []
        for g in range(_G):
            s, ts, idxv = carry[g]
            m = jnp.max(s)
            fidx = jnp.min(jnp.where(s == m, iflat, _NTOT))
            upd = i256 == k
            ts = jnp.where(upd, m, ts)
            idxv = jnp.where(upd, fidx, idxv)
            s = jnp.where(iflat == fidx, -1.0, s)
            out.append((s, ts, idxv))
        return tuple(out)

    picked = jax.lax.fori_loop(0, _PRE, pick, init)

    for g in range(_G):
        _, ts, idxv = picked[g]
        idxS = jnp.transpose(idxv)
        cidS = idxS // 128
        laneS = idxS % 128
        R = (cidS == i126).astype(jnp.float32)
        Lm = (laneS == i128).astype(jnp.float32)
        bS = []
        bL = []
        for j in range(4):
            m1 = jax.lax.dot_general(R, bx_ref[0, j], (((1,), (0,)), ((), ())),
                                     preferred_element_type=jnp.float32)
            bj = jnp.sum(m1 * Lm, axis=1, keepdims=True)
            bS.append(bj)
            bL.append(jnp.transpose(bj))
        b0S, b1S, b2S, b3S = bS
        b0L, b1L, b2L, b3L = bL
        areaS = jnp.maximum(b2S - b0S, 0.0) * jnp.maximum(b3S - b1S, 0.0)
        areaL = jnp.maximum(b2L - b0L, 0.0) * jnp.maximum(b3L - b1L, 0.0)
        for blk in range(4):
            sl = slice(blk * 64, blk * 64 + 64)
            ih = jnp.maximum(jnp.minimum(b2S[sl], b2L)
                             - jnp.maximum(b0S[sl], b0L), 0.0)
            iw = jnp.maximum(jnp.minimum(b3S[sl], b3L)
                             - jnp.maximum(b1S[sl], b1L), 0.0)
            inter = ih * iw
            union = jnp.maximum(areaS[sl] + areaL - inter, 1e-9)
            iou = inter / union
            isub = (jax.lax.broadcasted_iota(jnp.int32, (64, 1), 0)
                    + blk * 64)
            iou_ref[g, sl, :] = ((iou > _NMS_T) & (isub < i256)
                                 ).astype(jnp.float32)

        def cond(c):
            return c[1]

        def body(c):
            k, _ = c
            supp = jax.lax.dot_general(k, iou_ref[g], (((1,), (0,)), ((), ())),
                                       preferred_element_type=jnp.float32)
            kn = jnp.where(supp > 0.5, 0.0, 1.0)
            return kn, jnp.any(kn != k)

        keep, _ = jax.lax.while_loop(
            cond, body, (jnp.ones((1, _PRE), jnp.float32), True))
        os_ref[0, g] = ts * keep
        ob_ref[0, g] = jnp.concatenate([b0L, b1L, b2L, b3L], axis=0)


def _final_body(s_ref, b_ref, fb_ref, fs_ref, fc_ref, v_ref):
    i160 = jax.lax.broadcasted_iota(jnp.int32, (_FCH, 1), 0)
    i128 = jax.lax.broadcasted_iota(jnp.int32, (1, 128), 1)
    i100 = jax.lax.broadcasted_iota(jnp.int32, (1, _MAXB), 1)
    z100 = jnp.zeros((1, _MAXB), jnp.float32)
    cmax0 = jnp.max(s_ref[0], axis=1, keepdims=True)

    def pick(k, carry):
        cmax, fs, f0, f1, f2, f3, fc = carry
        m = jnp.max(cmax)
        cid = jnp.min(jnp.where(cmax == m, i160, _FCH))
        row = s_ref[0, pl.ds(cid, 1), :]
        lane = jnp.min(jnp.where(row == m, i128, 128))
        sel = i128 == lane
        flat = cid * 128 + lane
        cls = (flat // _PRE).astype(jnp.float32)

        def getb(j):
            br = b_ref[0, j, pl.ds(cid, 1), :]
            return jnp.sum(jnp.where(sel, br, 0.0))

        upd = i100 == k
        fs = jnp.where(upd, m, fs)
        f0 = jnp.where(upd, getb(0), f0)
        f1 = jnp.where(upd, getb(1), f1)
        f2 = jnp.where(upd, getb(2), f2)
        f3 = jnp.where(upd, getb(3), f3)
        fc = jnp.where(upd, cls, fc)
        nrow = jnp.where(sel, -1.0, row)
        s_ref[0, pl.ds(cid, 1), :] = nrow
        cmax = jnp.where(i160 == cid, jnp.max(nrow), cmax)
        return cmax, fs, f0, f1, f2, f3, fc

    init = (cmax0, z100, z100, z100, z100, z100, z100)
    _, fs, f0, f1, f2, f3, fc = jax.lax.fori_loop(0, _MAXB, pick, init)
    fs_ref[0] = fs
    fc_ref[0] = fc
    fb_ref[0] = jnp.concatenate([f0, f1, f2, f3], axis=0)
    v_ref[0] = jnp.sum((fs > 0.0).astype(jnp.int32), axis=1, keepdims=True)


def kernel(inputs_3, inputs_4, inputs_5):
    B = inputs_3.shape[0]
    xs = (inputs_3, inputs_4, inputs_5)
    s_all, b_all = [], []
    for x, (H, W, mask, stride, scale) in zip(xs, _LEVELS):
        s, b = _decode_level(x, H, W, mask, stride, scale)
        s_all.append(s)
        b_all.append(b)
    scores = jnp.concatenate(s_all, axis=2)
    boxes = jnp.concatenate(b_all, axis=2)
    scores = scores.reshape(B, _CLASSES, _NCH, 128)
    boxes = boxes.reshape(B, 4, _NCH, 128)

    out_s, out_b = pl.pallas_call(
        _nms_body,
        grid=(B, _CLASSES // _G),
        in_specs=[
            pl.BlockSpec((1, _G, _NCH, 128), lambda b, c: (b, c, 0, 0)),
            pl.BlockSpec((1, 4, _NCH, 128), lambda b, c: (b, 0, 0, 0)),
        ],
        out_specs=[
            pl.BlockSpec((1, _G, 1, _PRE), lambda b, c: (b, c, 0, 0)),
            pl.BlockSpec((1, _G, 4, _PRE), lambda b, c: (b, c, 0, 0)),
        ],
        out_shape=[
            jax.ShapeDtypeStruct((B, _CLASSES, 1, _PRE), jnp.float32),
            jax.ShapeDtypeStruct((B, _CLASSES, 4, _PRE), jnp.float32),
        ],
        scratch_shapes=[pltpu.VMEM((_G, _PRE, _PRE), jnp.float32)],
    )(scores, boxes)

    flat_s = out_s.reshape(B, _FCH, 128)
    flat_b = jnp.transpose(out_b, (0, 2, 1, 3)).reshape(B, 4, _FCH, 128)

    fb, fs, fc, valid = pl.pallas_call(
        _final_body,
        grid=(B,),
        in_specs=[
            pl.BlockSpec((1, _FCH, 128), lambda i: (i, 0, 0)),
            pl.BlockSpec((1, 4, _FCH, 128), lambda i: (i, 0, 0, 0)),
        ],
        out_specs=[
            pl.BlockSpec((1, 4, _MAXB), lambda i: (i, 0, 0)),
            pl.BlockSpec((1, 1, _MAXB), lambda i: (i, 0, 0)),
            pl.BlockSpec((1, 1, _MAXB), lambda i: (i, 0, 0)),
            pl.BlockSpec((1, 1, 1), lambda i: (i, 0, 0)),
        ],
        out_shape=[
            jax.ShapeDtypeStruct((B, 4, _MAXB), jnp.float32),
            jax.ShapeDtypeStruct((B, 1, _MAXB), jnp.float32),
            jax.ShapeDtypeStruct((B, 1, _MAXB), jnp.float32),
            jax.ShapeDtypeStruct((B, 1, 1), jnp.int32),
        ],
    )(flat_s, flat_b)

    fin_b = jnp.transpose(fb, (0, 2, 1))
    fin_s = fs.reshape(B, _MAXB)
    fin_c = fc.reshape(B, _MAXB)
    return fin_b, fin_s, fin_c, valid.reshape(B)


# speedup vs baseline: 2.2909x; 1.0986x over previous
import functools
import numpy as np
import jax
import jax.numpy as jnp
from jax.experimental import pallas as pl
from jax.experimental.pallas import tpu as pltpu

_CLASSES = 80
_NMS_T = 0.6
_PRE = 256
_MAXB = 100
_ANCH = np.array([[12, 16], [19, 36], [40, 28], [36, 75], [76, 55],
                  [72, 146], [142, 110], [192, 243], [459, 401]],
                 dtype=np.float32)
_LEVELS = (
    (64, 64, (0, 1, 2), 8.0, 1.2),
    (32, 32, (3, 4, 5), 16.0, 1.1),
    (16, 16, (6, 7, 8), 32.0, 1.05),
)
_NTOT = 16128
_NCH = 126
_FCH = 160


def _sig(x):
    return 1.0 / (1.0 + jnp.exp(-x))


def _decode_body(x_ref, s_ref, b_ref, *, H, W, anchors, stride, scale):
    HW = H * W
    ipos = jax.lax.broadcasted_iota(jnp.int32, (1, HW), 1)
    gx = (ipos % W).astype(jnp.float32)
    gy = (ipos // W).astype(jnp.float32)
    bias = 0.5 * (scale - 1.0)
    for a in range(3):
        base = a * 85
        head = x_ref[0, base:base + 5, :]
        cls = x_ref[0, base + 5:base + 85, :]
        tx, ty = head[0:1], head[1:2]
        tw, th = head[2:3], head[3:4]
        obj = head[4:5]
        xc = (_sig(tx) * scale - bias + gx) / W
        yc = (_sig(ty) * scale - bias + gy) / H
        w = jnp.exp(tw) * (anchors[a][0] / (stride * W))
        h = jnp.exp(th) * (anchors[a][1] / (stride * H))
        b_ref[0, :, a, :] = jnp.concatenate(
            [yc - h * 0.5, xc - w * 0.5, yc + h * 0.5, xc + w * 0.5], axis=0)
        s_ref[0, :, a, :] = _sig(cls) * _sig(obj)


def _decode_level(x, H, W, mask, stride, scale):
    B = x.shape[0]
    HW = H * W
    xr = jnp.transpose(x.reshape(B, HW, 255), (0, 2, 1))
    anchors = tuple((float(_ANCH[m, 0]), float(_ANCH[m, 1])) for m in mask)
    body = functools.partial(_decode_body, H=H, W=W, anchors=anchors,
                             stride=stride, scale=scale)
    s, b = pl.pallas_call(
        body,
        grid=(B,),
        in_specs=[pl.BlockSpec((1, 255, HW), lambda i: (i, 0, 0))],
        out_specs=[pl.BlockSpec((1, _CLASSES, 3, HW), lambda i: (i, 0, 0, 0)),
                   pl.BlockSpec((1, 4, 3, HW), lambda i: (i, 0, 0, 0))],
        out_shape=[jax.ShapeDtypeStruct((B, _CLASSES, 3, HW), jnp.float32),
                   jax.ShapeDtypeStruct((B, 4, 3, HW), jnp.float32)],
    )(xr)
    s = jnp.transpose(s, (0, 1, 3, 2)).reshape(B, _CLASSES, HW * 3)
    b = jnp.transpose(b, (0, 1, 3, 2)).reshape(B, 4, HW * 3)
    return s, b


_G = 4


def _nms_body(s_ref, bx_ref, os_ref, ob_ref, iou_ref):
    i126 = jax.lax.broadcasted_iota(jnp.int32, (1, _NCH), 1)
    i128 = jax.lax.broadcasted_iota(jnp.int32, (1, 128), 1)
    i256 = jax.lax.broadcasted_iota(jnp.int32, (1, _PRE), 1)
    iflat = (jax.lax.broadcasted_iota(jnp.int32, (_NCH, 128), 0) * 128
             + jax.lax.broadcasted_iota(jnp.int32, (_NCH, 128), 1))
    z256f = jnp.zeros((1, _PRE), jnp.float32)
    z256i = jnp.zeros((1, _PRE), jnp.int32)

    init = tuple((s_ref[0, g], z256f, z256i) for g in range(_G))

    def pick(k, carry):
        out = []
        for g in range(_G):
            s, ts, idxv = carry[g]
            m = jnp.max(s)
            fidx = jnp.min(jnp.where(s == m, iflat, _NTOT))
            upd = i256 == k
            ts = jnp.where(upd, m, ts)
            idxv = jnp.where(upd, fidx, idxv)
            s = jnp.where(iflat == fidx, -1.0, s)
            out.append((s, ts, idxv))
        return tuple(out)

    picked = jax.lax.fori_loop(0, _PRE, pick, init)

    for g in range(_G):
        _, ts, idxv = picked[g]
        idxS = jnp.transpose(idxv)
        cidS = idxS // 128
        laneS = idxS % 128
        R = (cidS == i126).astype(jnp.float32)
        Lm = (laneS == i128).astype(jnp.float32)
        bS = []
        bL = []
        for j in range(4):
            m1 = jax.lax.dot_general(R, bx_ref[0, j], (((1,), (0,)), ((), ())),
                                     preferred_element_type=jnp.float32)
            bj = jnp.sum(m1 * Lm, axis=1, keepdims=True)
            bS.append(bj)
            bL.append(jnp.transpose(bj))
        b0S, b1S, b2S, b3S = bS
        b0L, b1L, b2L, b3L = bL
        areaS = jnp.maximum(b2S - b0S, 0.0) * jnp.maximum(b3S - b1S, 0.0)
        areaL = jnp.maximum(b2L - b0L, 0.0) * jnp.maximum(b3L - b1L, 0.0)
        for blk in range(4):
            sl = slice(blk * 64, blk * 64 + 64)
            ih = jnp.maximum(jnp.minimum(b2S[sl], b2L)
                             - jnp.maximum(b0S[sl], b0L), 0.0)
            iw = jnp.maximum(jnp.minimum(b3S[sl], b3L)
                             - jnp.maximum(b1S[sl], b1L), 0.0)
            inter = ih * iw
            union = jnp.maximum(areaS[sl] + areaL - inter, 1e-9)
            iou = inter / union
            isub = (jax.lax.broadcasted_iota(jnp.int32, (64, 1), 0)
                    + blk * 64)
            iou_ref[g, sl, :] = ((iou > _NMS_T) & (isub < i256)
                                 ).astype(jnp.float32)

        def cond(c):
            return c[1]

        def body(c):
            k, _ = c
            supp = jax.lax.dot_general(k, iou_ref[g], (((1,), (0,)), ((), ())),
                                       preferred_element_type=jnp.float32)
            kn = jnp.where(supp > 0.5, 0.0, 1.0)
            return kn, jnp.any(kn != k)

        keep, _ = jax.lax.while_loop(
            cond, body, (jnp.ones((1, _PRE), jnp.float32), True))
        os_ref[0, g] = ts * keep
        ob_ref[0, g] = jnp.concatenate([b0L, b1L, b2L, b3L], axis=0)


def _final_body(s_ref, b_ref, fb_ref, fs_ref, fc_ref, v_ref):
    i160 = jax.lax.broadcasted_iota(jnp.int32, (_FCH, 1), 0)
    i128 = jax.lax.broadcasted_iota(jnp.int32, (1, 128), 1)
    i100 = jax.lax.broadcasted_iota(jnp.int32, (1, _MAXB), 1)
    z100 = jnp.zeros((1, _MAXB), jnp.float32)
    cmax0 = jnp.max(s_ref[0], axis=1, keepdims=True)

    def pick(k, carry):
        cmax, fs, f0, f1, f2, f3, fc = carry
        m = jnp.max(cmax)
        cid = jnp.min(jnp.where(cmax == m, i160, _FCH))
        row = s_ref[0, pl.ds(cid, 1), :]
        lane = jnp.min(jnp.where(row == m, i128, 128))
        sel = i128 == lane
        flat = cid * 128 + lane
        cls = (flat // _PRE).astype(jnp.float32)

        def getb(j):
            br = b_ref[0, j, pl.ds(cid, 1), :]
            return jnp.sum(jnp.where(sel, br, 0.0))

        upd = i100 == k
        fs = jnp.where(upd, m, fs)
        f0 = jnp.where(upd, getb(0), f0)
        f1 = jnp.where(upd, getb(1), f1)
        f2 = jnp.where(upd, getb(2), f2)
        f3 = jnp.where(upd, getb(3), f3)
        fc = jnp.where(upd, cls, fc)
        nrow = jnp.where(sel, -1.0, row)
        s_ref[0, pl.ds(cid, 1), :] = nrow
        cmax = jnp.where(i160 == cid, jnp.max(nrow), cmax)
        return cmax, fs, f0, f1, f2, f3, fc

    init = (cmax0, z100, z100, z100, z100, z100, z100)
    _, fs, f0, f1, f2, f3, fc = jax.lax.fori_loop(0, _MAXB, pick, init)
    fs_ref[0] = fs
    fc_ref[0] = fc
    fb_ref[0] = jnp.concatenate([f0, f1, f2, f3], axis=0)
    v_ref[0] = jnp.sum((fs > 0.0).astype(jnp.int32), axis=1, keepdims=True)


def kernel(inputs_3, inputs_4, inputs_5):
    B = inputs_3.shape[0]
    xs = (inputs_3, inputs_4, inputs_5)
    s_all, b_all = [], []
    for x, (H, W, mask, stride, scale) in zip(xs, _LEVELS):
        s, b = _decode_level(x, H, W, mask, stride, scale)
        s_all.append(s)
        b_all.append(b)
    scores = jnp.concatenate(s_all, axis=2)
    boxes = jnp.concatenate(b_all, axis=2)
    scores = scores.reshape(B, _CLASSES, _NCH, 128)
    boxes = boxes.reshape(B, 4, _NCH, 128)

    out_s, out_b = pl.pallas_call(
        _nms_body,
        grid=(B, _CLASSES // _G),
        in_specs=[
            pl.BlockSpec((1, _G, _NCH, 128), lambda b, c: (b, c, 0, 0)),
            pl.BlockSpec((1, 4, _NCH, 128), lambda b, c: (b, 0, 0, 0)),
        ],
        out_specs=[
            pl.BlockSpec((1, _G, 1, _PRE), lambda b, c: (b, c, 0, 0)),
            pl.BlockSpec((1, _G, 4, _PRE), lambda b, c: (b, c, 0, 0)),
        ],
        out_shape=[
            jax.ShapeDtypeStruct((B, _CLASSES, 1, _PRE), jnp.float32),
            jax.ShapeDtypeStruct((B, _CLASSES, 4, _PRE), jnp.float32),
        ],
        scratch_shapes=[pltpu.VMEM((_G, _PRE, _PRE), jnp.float32)],
    )(scores, boxes)

    flat_s = out_s.reshape(B, _FCH, 128)
    flat_b = jnp.transpose(out_b, (0, 2, 1, 3)).reshape(B, 4, _FCH, 128)

    fb, fs, fc, valid = pl.pallas_call(
        _final_body,
        grid=(B,),
        in_specs=[
            pl.BlockSpec((1, _FCH, 128), lambda i: (i, 0, 0)),
            pl.BlockSpec((1, 4, _FCH, 128), lambda i: (i, 0, 0, 0)),
        ],
        out_specs=[
            pl.BlockSpec((1, 4, _MAXB), lambda i: (i, 0, 0)),
            pl.BlockSpec((1, 1, _MAXB), lambda i: (i, 0, 0)),
            pl.BlockSpec((1, 1, _MAXB), lambda i: (i, 0, 0)),
            pl.BlockSpec((1, 1, 1), lambda i: (i, 0, 0)),
        ],
        out_shape=[
            jax.ShapeDtypeStruct((B, 4, _MAXB), jnp.float32),
            jax.ShapeDtypeStruct((B, 1, _MAXB), jnp.float32),
            jax.ShapeDtypeStruct((B, 1, _MAXB), jnp.float32),
            jax.ShapeDtypeStruct((B, 1, 1), jnp.int32),
        ],
    )(flat_s, flat_b)

    fin_b = jnp.transpose(fb, (0, 2, 1))
    fin_s = fs.reshape(B, _MAXB)
    fin_c = fc.reshape(B, _MAXB)
    return fin_b, fin_s, fin_c, valid.reshape(B)


# speedup vs baseline: 2.4053x; 1.0499x over previous
import functools
import numpy as np
import jax
import jax.numpy as jnp
from jax.experimental import pallas as pl
from jax.experimental.pallas import tpu as pltpu

_CLASSES = 80
_NMS_T = 0.6
_PRE = 256
_MAXB = 100
_ANCH = np.array([[12, 16], [19, 36], [40, 28], [36, 75], [76, 55],
                  [72, 146], [142, 110], [192, 243], [459, 401]],
                 dtype=np.float32)
_LEVELS = (
    (64, 64, (0, 1, 2), 8.0, 1.2),
    (32, 32, (3, 4, 5), 16.0, 1.1),
    (16, 16, (6, 7, 8), 32.0, 1.05),
)
_NTOT = 16128
_NCH = 126
_FCH = 160


def _sig(x):
    return 1.0 / (1.0 + jnp.exp(-x))


def _decode_body(x_ref, s_ref, b_ref, *, H, W, anchors, stride, scale):
    HW = H * W
    ipos = jax.lax.broadcasted_iota(jnp.int32, (1, HW), 1)
    gx = (ipos % W).astype(jnp.float32)
    gy = (ipos // W).astype(jnp.float32)
    bias = 0.5 * (scale - 1.0)
    for a in range(3):
        base = a * 85
        head = x_ref[0, base:base + 5, :]
        cls = x_ref[0, base + 5:base + 85, :]
        tx, ty = head[0:1], head[1:2]
        tw, th = head[2:3], head[3:4]
        obj = head[4:5]
        xc = (_sig(tx) * scale - bias + gx) / W
        yc = (_sig(ty) * scale - bias + gy) / H
        w = jnp.exp(tw) * (anchors[a][0] / (stride * W))
        h = jnp.exp(th) * (anchors[a][1] / (stride * H))
        b_ref[0, :, a, :] = jnp.concatenate(
            [yc - h * 0.5, xc - w * 0.5, yc + h * 0.5, xc + w * 0.5], axis=0)
        s_ref[0, :, a, :] = _sig(cls) * _sig(obj)


def _decode_level(x, H, W, mask, stride, scale):
    B = x.shape[0]
    HW = H * W
    xr = jnp.transpose(x.reshape(B, HW, 255), (0, 2, 1))
    anchors = tuple((float(_ANCH[m, 0]), float(_ANCH[m, 1])) for m in mask)
    body = functools.partial(_decode_body, H=H, W=W, anchors=anchors,
                             stride=stride, scale=scale)
    s, b = pl.pallas_call(
        body,
        grid=(B,),
        in_specs=[pl.BlockSpec((1, 255, HW), lambda i: (i, 0, 0))],
        out_specs=[pl.BlockSpec((1, _CLASSES, 3, HW), lambda i: (i, 0, 0, 0)),
                   pl.BlockSpec((1, 4, 3, HW), lambda i: (i, 0, 0, 0))],
        out_shape=[jax.ShapeDtypeStruct((B, _CLASSES, 3, HW), jnp.float32),
                   jax.ShapeDtypeStruct((B, 4, 3, HW), jnp.float32)],
    )(xr)
    s = jnp.transpose(s, (0, 1, 3, 2)).reshape(B, _CLASSES, HW * 3)
    b = jnp.transpose(b, (0, 1, 3, 2)).reshape(B, 4, HW * 3)
    return s, b


_G = 8


def _nms_body(s_ref, bx_ref, os_ref, ob_ref, iou_ref):
    i126 = jax.lax.broadcasted_iota(jnp.int32, (1, _NCH), 1)
    i128 = jax.lax.broadcasted_iota(jnp.int32, (1, 128), 1)
    i256 = jax.lax.broadcasted_iota(jnp.int32, (1, _PRE), 1)
    iflat = (jax.lax.broadcasted_iota(jnp.int32, (_NCH, 128), 0) * 128
             + jax.lax.broadcasted_iota(jnp.int32, (_NCH, 128), 1))
    z256f = jnp.zeros((1, _PRE), jnp.float32)
    z256i = jnp.zeros((1, _PRE), jnp.int32)

    init = tuple((s_ref[0, g], z256f, z256i) for g in range(_G))

    def pick(k, carry):
        out = []
        for g in range(_G):
            s, ts, idxv = carry[g]
            m = jnp.max(s)
            fidx = jnp.min(jnp.where(s == m, iflat, _NTOT))
            upd = i256 == k
            ts = jnp.where(upd, m, ts)
            idxv = jnp.where(upd, fidx, idxv)
            s = jnp.where(iflat == fidx, -1.0, s)
            out.append((s, ts, idxv))
        return tuple(out)

    picked = jax.lax.fori_loop(0, _PRE, pick, init)

    for g in range(_G):
        _, ts, idxv = picked[g]
        idxS = jnp.transpose(idxv)
        cidS = idxS // 128
        laneS = idxS % 128
        R = (cidS == i126).astype(jnp.float32)
        Lm = (laneS == i128).astype(jnp.float32)
        bS = []
        bL = []
        for j in range(4):
            m1 = jax.lax.dot_general(R, bx_ref[0, j], (((1,), (0,)), ((), ())),
                                     preferred_element_type=jnp.float32)
            bj = jnp.sum(m1 * Lm, axis=1, keepdims=True)
            bS.append(bj)
            bL.append(jnp.transpose(bj))
        b0S, b1S, b2S, b3S = bS
        b0L, b1L, b2L, b3L = bL
        areaS = jnp.maximum(b2S - b0S, 0.0) * jnp.maximum(b3S - b1S, 0.0)
        areaL = jnp.maximum(b2L - b0L, 0.0) * jnp.maximum(b3L - b1L, 0.0)
        for blk in range(4):
            sl = slice(blk * 64, blk * 64 + 64)
            ih = jnp.maximum(jnp.minimum(b2S[sl], b2L)
                             - jnp.maximum(b0S[sl], b0L), 0.0)
            iw = jnp.maximum(jnp.minimum(b3S[sl], b3L)
                             - jnp.maximum(b1S[sl], b1L), 0.0)
            inter = ih * iw
            union = jnp.maximum(areaS[sl] + areaL - inter, 1e-9)
            iou = inter / union
            isub = (jax.lax.broadcasted_iota(jnp.int32, (64, 1), 0)
                    + blk * 64)
            iou_ref[g, sl, :] = ((iou > _NMS_T) & (isub < i256)
                                 ).astype(jnp.float32)

        def cond(c):
            return c[1]

        def body(c):
            k, _ = c
            supp = jax.lax.dot_general(k, iou_ref[g], (((1,), (0,)), ((), ())),
                                       preferred_element_type=jnp.float32)
            kn = jnp.where(supp > 0.5, 0.0, 1.0)
            return kn, jnp.any(kn != k)

        keep, _ = jax.lax.while_loop(
            cond, body, (jnp.ones((1, _PRE), jnp.float32), True))
        os_ref[0, g] = ts * keep
        ob_ref[0, g] = jnp.concatenate([b0L, b1L, b2L, b3L], axis=0)


def _final_body(s_ref, b_ref, fb_ref, fs_ref, fc_ref, v_ref):
    i160 = jax.lax.broadcasted_iota(jnp.int32, (_FCH, 1), 0)
    i128 = jax.lax.broadcasted_iota(jnp.int32, (1, 128), 1)
    i100 = jax.lax.broadcasted_iota(jnp.int32, (1, _MAXB), 1)
    z100 = jnp.zeros((1, _MAXB), jnp.float32)
    cmax0 = jnp.max(s_ref[0], axis=1, keepdims=True)

    def pick(k, carry):
        cmax, fs, f0, f1, f2, f3, fc = carry
        m = jnp.max(cmax)
        cid = jnp.min(jnp.where(cmax == m, i160, _FCH))
        row = s_ref[0, pl.ds(cid, 1), :]
        lane = jnp.min(jnp.where(row == m, i128, 128))
        sel = i128 == lane
        flat = cid * 128 + lane
        cls = (flat // _PRE).astype(jnp.float32)

        def getb(j):
            br = b_ref[0, j, pl.ds(cid, 1), :]
            return jnp.sum(jnp.where(sel, br, 0.0))

        upd = i100 == k
        fs = jnp.where(upd, m, fs)
        f0 = jnp.where(upd, getb(0), f0)
        f1 = jnp.where(upd, getb(1), f1)
        f2 = jnp.where(upd, getb(2), f2)
        f3 = jnp.where(upd, getb(3), f3)
        fc = jnp.where(upd, cls, fc)
        nrow = jnp.where(sel, -1.0, row)
        s_ref[0, pl.ds(cid, 1), :] = nrow
        cmax = jnp.where(i160 == cid, jnp.max(nrow), cmax)
        return cmax, fs, f0, f1, f2, f3, fc

    init = (cmax0, z100, z100, z100, z100, z100, z100)
    _, fs, f0, f1, f2, f3, fc = jax.lax.fori_loop(0, _MAXB, pick, init)
    fs_ref[0] = fs
    fc_ref[0] = fc
    fb_ref[0] = jnp.concatenate([f0, f1, f2, f3], axis=0)
    v_ref[0] = jnp.sum((fs > 0.0).astype(jnp.int32), axis=1, keepdims=True)


def kernel(inputs_3, inputs_4, inputs_5):
    B = inputs_3.shape[0]
    xs = (inputs_3, inputs_4, inputs_5)
    s_all, b_all = [], []
    for x, (H, W, mask, stride, scale) in zip(xs, _LEVELS):
        s, b = _decode_level(x, H, W, mask, stride, scale)
        s_all.append(s)
        b_all.append(b)
    scores = jnp.concatenate(s_all, axis=2)
    boxes = jnp.concatenate(b_all, axis=2)
    scores = scores.reshape(B, _CLASSES, _NCH, 128)
    boxes = boxes.reshape(B, 4, _NCH, 128)

    out_s, out_b = pl.pallas_call(
        _nms_body,
        grid=(B, _CLASSES // _G),
        in_specs=[
            pl.BlockSpec((1, _G, _NCH, 128), lambda b, c: (b, c, 0, 0)),
            pl.BlockSpec((1, 4, _NCH, 128), lambda b, c: (b, 0, 0, 0)),
        ],
        out_specs=[
            pl.BlockSpec((1, _G, 1, _PRE), lambda b, c: (b, c, 0, 0)),
            pl.BlockSpec((1, _G, 4, _PRE), lambda b, c: (b, c, 0, 0)),
        ],
        out_shape=[
            jax.ShapeDtypeStruct((B, _CLASSES, 1, _PRE), jnp.float32),
            jax.ShapeDtypeStruct((B, _CLASSES, 4, _PRE), jnp.float32),
        ],
        scratch_shapes=[pltpu.VMEM((_G, _PRE, _PRE), jnp.float32)],
    )(scores, boxes)

    flat_s = out_s.reshape(B, _FCH, 128)
    flat_b = jnp.transpose(out_b, (0, 2, 1, 3)).reshape(B, 4, _FCH, 128)

    fb, fs, fc, valid = pl.pallas_call(
        _final_body,
        grid=(B,),
        in_specs=[
            pl.BlockSpec((1, _FCH, 128), lambda i: (i, 0, 0)),
            pl.BlockSpec((1, 4, _FCH, 128), lambda i: (i, 0, 0, 0)),
        ],
        out_specs=[
            pl.BlockSpec((1, 4, _MAXB), lambda i: (i, 0, 0)),
            pl.BlockSpec((1, 1, _MAXB), lambda i: (i, 0, 0)),
            pl.BlockSpec((1, 1, _MAXB), lambda i: (i, 0, 0)),
            pl.BlockSpec((1, 1, 1), lambda i: (i, 0, 0)),
        ],
        out_shape=[
            jax.ShapeDtypeStruct((B, 4, _MAXB), jnp.float32),
            jax.ShapeDtypeStruct((B, 1, _MAXB), jnp.float32),
            jax.ShapeDtypeStruct((B, 1, _MAXB), jnp.float32),
            jax.ShapeDtypeStruct((B, 1, 1), jnp.int32),
        ],
    )(flat_s, flat_b)

    fin_b = jnp.transpose(fb, (0, 2, 1))
    fin_s = fs.reshape(B, _MAXB)
    fin_c = fc.reshape(B, _MAXB)
    return fin_b, fin_s, fin_c, valid.reshape(B)
